# baseline (speedup 1.0000x reference)
import functools

import jax
import jax.numpy as jnp
from jax import lax
from jax.experimental import pallas as pl
from jax.experimental.pallas import tpu as pltpu

_BS = 8
_BL = 2048
_LANE = 128


def _combine_tables(special_table, blosum_table):
    n_special = special_table.shape[0]
    v, d = blosum_table.shape

    def body(sp_ref, bl_ref, out_ref):
        bl = bl_ref[...]
        rid = lax.broadcasted_iota(jnp.int32, bl.shape, 0)
        ext = jnp.broadcast_to(sp_ref[0:1], bl.shape)
        for r in range(1, n_special):
            row = jnp.broadcast_to(sp_ref[r:r + 1], bl.shape)
            ext = jnp.where(rid == r, row, ext)
        comb = bl + ext
        comb_t = jnp.concatenate(
            [comb.T, jnp.zeros((d, _LANE - v), jnp.float32)], axis=1)
        for dd in range(d):
            row = comb_t[dd:dd + 1, :]
            out_ref[dd] = jnp.broadcast_to(row, (8, _LANE))

    return pl.pallas_call(
        body,
        out_shape=jax.ShapeDtypeStruct((d, 8, _LANE), jnp.float32),
    )(special_table, blosum_table)


def _gather_physical(lut, idx_t):
    d = lut.shape[0]
    s, b = idx_t.shape

    def body(lut_ref, idx_ref, out_ref):
        for c in range(_BL // _LANE):
            t = idx_ref[:, c * _LANE:(c + 1) * _LANE]
            for dd in range(d):
                res = jnp.take_along_axis(
                    lut_ref[dd], t, axis=1,
                    mode=lax.GatherScatterMode.PROMISE_IN_BOUNDS)
                out_ref[dd, :, c * _LANE:(c + 1) * _LANE] = res

    return pl.pallas_call(
        body,
        grid=(s // _BS, b // _BL),
        in_specs=[
            pl.BlockSpec((d, 8, _LANE), lambda i, j: (0, 0, 0)),
            pl.BlockSpec((_BS, _BL), lambda i, j: (i, j)),
        ],
        out_specs=pl.BlockSpec((d, _BS, _BL), lambda i, j: (0, i, j)),
        out_shape=jax.ShapeDtypeStruct((d, s, b), jnp.float32),
    )(lut, idx_t)


def kernel(token_indices, special_table, blosum_table):
    lut = _combine_tables(special_table, blosum_table)
    idx_t = token_indices.astype(jnp.int32).T
    out_t = _gather_physical(lut, idx_t)
    return out_t.transpose(2, 1, 0)


# speedup vs baseline: 1.5205x; 1.5205x over previous
import functools

import jax
import jax.numpy as jnp
from jax import lax
from jax.experimental import pallas as pl
from jax.experimental.pallas import tpu as pltpu

_BS = 8
_BL = 4096
_LANE = 128


def _combine_tables(special_table, blosum_table):
    n_special = special_table.shape[0]
    v, d = blosum_table.shape

    def body(sp_ref, bl_ref, out_ref):
        bl = bl_ref[...]
        rid = lax.broadcasted_iota(jnp.int32, bl.shape, 0)
        ext = jnp.broadcast_to(sp_ref[0:1], bl.shape)
        for r in range(1, n_special):
            row = jnp.broadcast_to(sp_ref[r:r + 1], bl.shape)
            ext = jnp.where(rid == r, row, ext)
        comb = bl + ext
        comb = jnp.concatenate(
            [comb, jnp.zeros((32 - v, d), jnp.float32)], axis=0)
        for dd in range(d):
            col = comb[:, dd:dd + 1]
            out_ref[dd] = jnp.broadcast_to(col, (32, _LANE))

    return pl.pallas_call(
        body,
        out_shape=jax.ShapeDtypeStruct((d, 32, _LANE), jnp.float32),
    )(special_table, blosum_table)


def _gather_physical(lut, idx_t):
    d = lut.shape[0]
    s, b = idx_t.shape

    def body(lut_ref, idx_ref, out_ref):
        for c in range(_BL // _LANE):
            t = idx_ref[:, c * _LANE:(c + 1) * _LANE]
            tl = jnp.bitwise_and(t, 7)
            hi0 = t < 8
            hi1 = t < 16
            for dd in range(d):
                gs = [
                    jnp.take_along_axis(
                        lut_ref[dd, 8 * g:8 * (g + 1)], tl, axis=0,
                        mode=lax.GatherScatterMode.PROMISE_IN_BOUNDS)
                    for g in range(3)
                ]
                res = jnp.where(hi0, gs[0], jnp.where(hi1, gs[1], gs[2]))
                out_ref[dd, :, c * _LANE:(c + 1) * _LANE] = res

    return pl.pallas_call(
        body,
        grid=(s // _BS, b // _BL),
        in_specs=[
            pl.BlockSpec((d, 32, _LANE), lambda i, j: (0, 0, 0)),
            pl.BlockSpec((_BS, _BL), lambda i, j: (i, j)),
        ],
        out_specs=pl.BlockSpec((d, _BS, _BL), lambda i, j: (0, i, j)),
        out_shape=jax.ShapeDtypeStruct((d, s, b), jnp.float32),
    )(lut, idx_t)


def kernel(token_indices, special_table, blosum_table):
    lut = _combine_tables(special_table, blosum_table)
    idx_t = token_indices.astype(jnp.int32).T
    out_t = _gather_physical(lut, idx_t)
    return out_t.transpose(2, 1, 0)


# speedup vs baseline: 1.9171x; 1.2609x over previous
import functools

import jax
import jax.numpy as jnp
from jax import lax
from jax.experimental import pallas as pl
from jax.experimental.pallas import tpu as pltpu

_BS = 8
_BL = 8192
_LANE = 128


def _combine_tables(special_table, blosum_table):
    n_special = special_table.shape[0]
    v, d = blosum_table.shape

    def body(sp_ref, bl_ref, out_ref):
        bl = bl_ref[...]
        rid = lax.broadcasted_iota(jnp.int32, bl.shape, 0)
        ext = jnp.broadcast_to(sp_ref[0:1], bl.shape)
        for r in range(1, n_special):
            row = jnp.broadcast_to(sp_ref[r:r + 1], bl.shape)
            ext = jnp.where(rid == r, row, ext)
        comb = bl + ext
        comb = jnp.concatenate(
            [comb, jnp.zeros((32 - v, d), jnp.float32)], axis=0)
        for dd in range(d):
            col = comb[:, dd:dd + 1]
            out_ref[dd] = jnp.broadcast_to(col, (32, _LANE))

    return pl.pallas_call(
        body,
        out_shape=jax.ShapeDtypeStruct((d, 32, _LANE), jnp.float32),
    )(special_table, blosum_table)


def _gather_physical(lut, idx_t):
    d = lut.shape[0]
    s, b = idx_t.shape

    def body(lut_ref, idx_ref, out_ref):
        for c in range(_BL // _LANE):
            t = idx_ref[:, c * _LANE:(c + 1) * _LANE]
            tl = jnp.bitwise_and(t, 7)
            hi0 = t < 8
            hi1 = t < 16
            for dd in range(d):
                gs = [
                    jnp.take_along_axis(
                        lut_ref[dd, 8 * g:8 * (g + 1)], tl, axis=0,
                        mode=lax.GatherScatterMode.PROMISE_IN_BOUNDS)
                    for g in range(3)
                ]
                res = jnp.where(hi0, gs[0], jnp.where(hi1, gs[1], gs[2]))
                out_ref[dd, :, c * _LANE:(c + 1) * _LANE] = res

    return pl.pallas_call(
        body,
        grid=(s // _BS, b // _BL),
        in_specs=[
            pl.BlockSpec((d, 32, _LANE), lambda i, j: (0, 0, 0)),
            pl.BlockSpec((_BS, _BL), lambda i, j: (i, j)),
        ],
        out_specs=pl.BlockSpec((d, _BS, _BL), lambda i, j: (0, i, j)),
        out_shape=jax.ShapeDtypeStruct((d, s, b), jnp.float32),
    )(lut, idx_t)


def kernel(token_indices, special_table, blosum_table):
    lut = _combine_tables(special_table, blosum_table)
    idx_t = token_indices.astype(jnp.int32).T
    out_t = _gather_physical(lut, idx_t)
    return out_t.transpose(2, 1, 0)


# speedup vs baseline: 2.1043x; 1.0976x over previous
import functools

import jax
import jax.numpy as jnp
from jax import lax
from jax.experimental import pallas as pl
from jax.experimental.pallas import tpu as pltpu

_BS = 8
_BL = 16384
_LANE = 128


def _combine_tables(special_table, blosum_table):
    n_special = special_table.shape[0]
    v, d = blosum_table.shape

    def body(sp_ref, bl_ref, out_ref):
        bl = bl_ref[...]
        rid = lax.broadcasted_iota(jnp.int32, bl.shape, 0)
        ext = jnp.broadcast_to(sp_ref[0:1], bl.shape)
        for r in range(1, n_special):
            row = jnp.broadcast_to(sp_ref[r:r + 1], bl.shape)
            ext = jnp.where(rid == r, row, ext)
        comb = bl + ext
        comb = jnp.concatenate(
            [comb, jnp.zeros((32 - v, d), jnp.float32)], axis=0)
        for dd in range(d):
            col = comb[:, dd:dd + 1]
            out_ref[dd] = jnp.broadcast_to(col, (32, _LANE))

    return pl.pallas_call(
        body,
        out_shape=jax.ShapeDtypeStruct((d, 32, _LANE), jnp.float32),
    )(special_table, blosum_table)


def _gather_physical(lut, idx_t):
    d = lut.shape[0]
    s, b = idx_t.shape

    def body(lut_ref, idx_ref, out_ref):
        for c in range(_BL // _LANE):
            t = idx_ref[:, c * _LANE:(c + 1) * _LANE]
            tl = jnp.bitwise_and(t, 7)
            hi0 = t < 8
            hi1 = t < 16
            for dd in range(d):
                gs = [
                    jnp.take_along_axis(
                        lut_ref[dd, 8 * g:8 * (g + 1)], tl, axis=0,
                        mode=lax.GatherScatterMode.PROMISE_IN_BOUNDS)
                    for g in range(3)
                ]
                res = jnp.where(hi0, gs[0], jnp.where(hi1, gs[1], gs[2]))
                out_ref[dd, :, c * _LANE:(c + 1) * _LANE] = res

    return pl.pallas_call(
        body,
        grid=(s // _BS, b // _BL),
        in_specs=[
            pl.BlockSpec((d, 32, _LANE), lambda i, j: (0, 0, 0)),
            pl.BlockSpec((_BS, _BL), lambda i, j: (i, j)),
        ],
        out_specs=pl.BlockSpec((d, _BS, _BL), lambda i, j: (0, i, j)),
        out_shape=jax.ShapeDtypeStruct((d, s, b), jnp.float32),
    )(lut, idx_t)


def kernel(token_indices, special_table, blosum_table):
    lut = _combine_tables(special_table, blosum_table)
    idx_t = token_indices.astype(jnp.int32).T
    out_t = _gather_physical(lut, idx_t)
    return out_t.transpose(2, 1, 0)
